# baseline (speedup 1.0000x reference)
import jax
import jax.numpy as jnp
from jax import lax
from jax.experimental import pallas as pl
from jax.experimental.pallas import tpu as pltpu
from jax.experimental.pallas import tpu_sc as plsc

DIM = 64
VOCAB = 1000000
BATCH = 4096
HIST = 50

L = 16
NW = 32
TOTAL = BATCH * HIST
IDX_COLS = 128
IDX_ROWS = TOTAL // IDX_COLS
ROWS_PER_W = IDX_ROWS // NW


def _body(idx_hbm, kern_hbm, table_hbm, out_hbm,
          idx_v, idxp_v, gat_v, rows_v, kern_v, gsem):
    wid = lax.axis_index("s") * 2 + lax.axis_index("c")
    pltpu.sync_copy(kern_hbm, kern_v)

    def step(m, carry):
        r = wid * ROWS_PER_W + m
        pltpu.sync_copy(idx_hbm.at[r], idx_v)

        def adj(k, c):
            v = idx_v[pl.ds(k * L, L)]
            vc = jnp.maximum(v - 2, 0)
            idxp_v[pl.ds(k * L, L)] = lax.shift_right_logical(vc, 1)
            return c
        lax.fori_loop(0, IDX_COLS // L, adj, 0)

        pltpu.async_copy(table_hbm.at[idxp_v], gat_v, gsem).wait()

        def sel(k, c):
            v = idx_v[pl.ds(k * L, L)]
            vc = jnp.maximum(v - 2, 0)
            off = (vc & 1) * DIM
            row_ids = k * L + lax.iota(jnp.int32, L)

            def col(d, cc):
                dv = jnp.full((L,), 0, jnp.int32) + d
                vals = plsc.load_gather(gat_v, (row_ids, off + dv))
                plsc.store_scatter(rows_v, (row_ids, dv), vals)
                return cc
            lax.fori_loop(0, DIM, col, 0, unroll=8)

            @pl.when(jnp.any(v < 2))
            def _patch():
                msk = v < 2
                cidx = jnp.minimum(v, 1)

                def pcol(d, cc):
                    dv = jnp.full((L,), 0, jnp.int32) + d
                    vals = plsc.load_gather(kern_v, (cidx, dv))
                    plsc.store_scatter(rows_v, (row_ids, dv), vals, mask=msk)
                    return cc
                lax.fori_loop(0, DIM, pcol, 0)
            return c
        lax.fori_loop(0, IDX_COLS // L, sel, 0)

        pltpu.sync_copy(rows_v, out_hbm.at[pl.ds(r * IDX_COLS, IDX_COLS)])
        return carry
    lax.fori_loop(0, ROWS_PER_W, step, 0)


@jax.jit
def kernel(inputs, kernel, priori):
    idx = inputs.reshape(-1).astype(jnp.int32).reshape(IDX_ROWS, IDX_COLS)
    table = priori.reshape(VOCAB // 2, 2 * DIM)
    mesh = plsc.VectorSubcoreMesh(core_axis_name="c", subcore_axis_name="s")
    k = pl.kernel(
        _body,
        out_type=jax.ShapeDtypeStruct((TOTAL, DIM), jnp.float32),
        mesh=mesh,
        compiler_params=pltpu.CompilerParams(
            needs_layout_passes=False, use_tc_tiling_on_sc=False),
        scratch_types=[
            pltpu.VMEM((IDX_COLS,), jnp.int32),
            pltpu.VMEM((IDX_COLS,), jnp.int32),
            pltpu.VMEM((IDX_COLS, 2 * DIM), jnp.float32),
            pltpu.VMEM((IDX_COLS, DIM), jnp.float32),
            pltpu.VMEM((2, DIM), jnp.float32),
            pltpu.SemaphoreType.DMA,
        ],
    )
    out = k(idx, kernel, table)
    return out.reshape(BATCH, HIST, DIM)


# speedup vs baseline: 1.5550x; 1.5550x over previous
import jax
import jax.numpy as jnp
from jax import lax
from jax.experimental import pallas as pl
from jax.experimental.pallas import tpu as pltpu
from jax.experimental.pallas import tpu_sc as plsc

DIM = 64
VOCAB = 1000000
BATCH = 4096
HIST = 50

L = 16
NW = 32
TOTAL = BATCH * HIST
CHUNK = 128
IDX_ROWS = TOTAL // CHUNK
ROWS_PER_W = IDX_ROWS // NW
HDIM = 32


def _body(idx_hbm, kern_hbm, table_hbm, out_hbm,
          idx_v, idxp_v, gat_v, kern_v, gsem):
    wid = lax.axis_index("s") * 2 + lax.axis_index("c")
    pltpu.sync_copy(kern_hbm, kern_v)
    lanes = lax.iota(jnp.int32, L)

    def step(m, carry):
        r = wid * ROWS_PER_W + m
        pltpu.sync_copy(idx_hbm.at[r], idx_v)

        def adj(g, c):
            v = idx_v[pl.ds(g * L, L)]
            e = lax.shift_left(jnp.maximum(v - 2, 0), 1)
            p = 2 * L * g + 2 * lanes
            plsc.store_scatter(
                idxp_v, (lax.shift_right_logical(p, 7), p & 127), e)
            q = p + 1
            plsc.store_scatter(
                idxp_v, (lax.shift_right_logical(q, 7), q & 127), e + 1)
            return c
        lax.fori_loop(0, CHUNK // L, adj, 0)

        h0 = pltpu.async_copy(
            table_hbm.at[idxp_v.at[0]], gat_v.at[pl.ds(0, CHUNK)], gsem)
        h1 = pltpu.async_copy(
            table_hbm.at[idxp_v.at[1]], gat_v.at[pl.ds(CHUNK, CHUNK)], gsem)
        h0.wait()
        h1.wait()

        def patch(g, c):
            v = idx_v[pl.ds(g * L, L)]

            @pl.when(jnp.any(v < 2))
            def _patch():
                msk = v < 2
                c2 = 2 * jnp.minimum(v, 1)
                rows2 = 2 * (g * L + lanes)

                def col(d, cc):
                    dv = jnp.full((L,), 0, jnp.int32) + d
                    ve = plsc.load_gather(kern_v, (c2, dv))
                    plsc.store_scatter(gat_v, (rows2, dv), ve, mask=msk)
                    vo = plsc.load_gather(kern_v, (c2 + 1, dv))
                    plsc.store_scatter(gat_v, (rows2 + 1, dv), vo, mask=msk)
                    return cc
                lax.fori_loop(0, HDIM, col, 0)
            return c
        lax.fori_loop(0, CHUNK // L, patch, 0)

        pltpu.sync_copy(gat_v, out_hbm.at[pl.ds(r * 2 * CHUNK, 2 * CHUNK)])
        return carry
    lax.fori_loop(0, ROWS_PER_W, step, 0)


@jax.jit
def kernel(inputs, kernel, priori):
    idx = inputs.reshape(-1).astype(jnp.int32).reshape(IDX_ROWS, CHUNK)
    table = priori.reshape(2 * VOCAB, HDIM)
    kern4 = kernel.reshape(4, HDIM)
    mesh = plsc.VectorSubcoreMesh(core_axis_name="c", subcore_axis_name="s")
    k = pl.kernel(
        _body,
        out_type=jax.ShapeDtypeStruct((2 * TOTAL, HDIM), jnp.float32),
        mesh=mesh,
        compiler_params=pltpu.CompilerParams(
            needs_layout_passes=False, use_tc_tiling_on_sc=False),
        scratch_types=[
            pltpu.VMEM((CHUNK,), jnp.int32),
            pltpu.VMEM((2, CHUNK), jnp.int32),
            pltpu.VMEM((2 * CHUNK, HDIM), jnp.float32),
            pltpu.VMEM((4, HDIM), jnp.float32),
            pltpu.SemaphoreType.DMA,
        ],
    )
    out = k(idx, kern4, table)
    return out.reshape(BATCH, HIST, DIM)


# speedup vs baseline: 1.6854x; 1.0839x over previous
import jax
import jax.numpy as jnp
from jax import lax
from jax.experimental import pallas as pl
from jax.experimental.pallas import tpu as pltpu
from jax.experimental.pallas import tpu_sc as plsc

DIM = 64
VOCAB = 1000000
BATCH = 4096
HIST = 50

L = 16
NW = 32
TOTAL = BATCH * HIST
CHUNK = 128
IDX_ROWS = TOTAL // CHUNK
ROWS_PER_W = IDX_ROWS // NW


def _body(idx_hbm, kern_hbm, table_hbm, out_hbm,
          idx_v, idxp_v, gat_v, kern_v, gsem):
    wid = lax.axis_index("s") * 2 + lax.axis_index("c")
    pltpu.sync_copy(kern_hbm, kern_v)
    lanes = lax.iota(jnp.int32, L)

    def step(m, carry):
        r = wid * ROWS_PER_W + m
        pltpu.sync_copy(idx_hbm.at[r], idx_v)

        def adj(g, c):
            v = idx_v[pl.ds(g * L, L)]
            idxp_v[pl.ds(g * L, L)] = lax.shift_left(
                jnp.maximum(v - 2, 0), 1)
            return c
        lax.fori_loop(0, CHUNK // L, adj, 0)

        pltpu.async_copy(table_hbm.at[idxp_v], gat_v, gsem).wait()

        def patch(g, c):
            v = idx_v[pl.ds(g * L, L)]

            @pl.when(jnp.any(v < 2))
            def _patch():
                msk = v < 2
                cidx = jnp.minimum(v, 1)
                rows = g * L + lanes

                def col(d, cc):
                    dv = jnp.full((L,), 0, jnp.int32) + d
                    ve = plsc.load_gather(kern_v, (cidx, dv))
                    plsc.store_scatter(gat_v, (rows, dv), ve, mask=msk)
                    return cc
                lax.fori_loop(0, DIM, col, 0)
            return c
        lax.fori_loop(0, CHUNK // L, patch, 0)

        pltpu.sync_copy(gat_v, out_hbm.at[pl.ds(r * CHUNK, CHUNK)])
        return carry
    lax.fori_loop(0, ROWS_PER_W, step, 0)


@jax.jit
def kernel(inputs, kernel, priori):
    idx = inputs.reshape(-1).astype(jnp.int32).reshape(IDX_ROWS, CHUNK)
    table = jnp.pad(priori, ((0, 0), (0, DIM))).reshape(2 * VOCAB, DIM)
    mesh = plsc.VectorSubcoreMesh(core_axis_name="c", subcore_axis_name="s")
    k = pl.kernel(
        _body,
        out_type=jax.ShapeDtypeStruct((TOTAL, DIM), jnp.float32),
        mesh=mesh,
        compiler_params=pltpu.CompilerParams(
            needs_layout_passes=False, use_tc_tiling_on_sc=False),
        scratch_types=[
            pltpu.VMEM((CHUNK,), jnp.int32),
            pltpu.VMEM((CHUNK,), jnp.int32),
            pltpu.VMEM((CHUNK, DIM), jnp.float32),
            pltpu.VMEM((2, DIM), jnp.float32),
            pltpu.SemaphoreType.DMA,
        ],
    )
    out = k(idx, kernel, table)
    return out.reshape(BATCH, HIST, DIM)
